# baseline (speedup 1.0000x reference)
import functools

import jax
import jax.numpy as jnp
from jax import lax
from jax.experimental import pallas as pl
from jax.experimental.pallas import tpu as pltpu
from jax.experimental.pallas import tpu_sc as plsc

NC = 2
NS = 16
NW = NC * NS


def _degree_counts(dst3, hi_rows):
    nblk, _, be = dst3.shape

    def body(dst_ref, m_ref):
        @pl.when(pl.program_id(0) == 0)
        def _():
            m_ref[...] = jnp.zeros_like(m_ref)

        d = dst_ref[...].reshape(1, be)
        oh_hi = (d // 128 == lax.broadcasted_iota(jnp.int32, (hi_rows, be), 0)
                 ).astype(jnp.bfloat16)
        oh_lo = (d % 128 == lax.broadcasted_iota(jnp.int32, (128, be), 0)
                 ).astype(jnp.bfloat16)
        m_ref[...] += lax.dot_general(oh_hi, oh_lo, (((1,), (1,)), ((), ())),
                                      preferred_element_type=jnp.float32)

    return pl.pallas_call(
        body,
        grid=(nblk,),
        in_specs=[pl.BlockSpec((1, 1, be), lambda i: (i, 0, 0))],
        out_specs=pl.BlockSpec((hi_rows, 128), lambda i: (0, 0)),
        out_shape=jax.ShapeDtypeStruct((hi_rows, 128), jnp.float32),
    )(dst3)


def _embed_stage(deg_col, idx_col, emb, W1, nb):
    n = deg_col.shape[0]
    v, d = emb.shape
    h = W1.shape[1]

    def body(deg_ref, idx_ref, emb_ref, w1_ref, dis_ref, x0_ref, y1_ref):
        deg = deg_ref[...]
        dis = jnp.where(deg > 0, lax.rsqrt(jnp.where(deg > 0, deg, 1.0)), 0.0)
        dis_ref[...] = dis
        oh = (idx_ref[...] == lax.broadcasted_iota(jnp.int32, (1, v), 1)
              ).astype(jnp.float32)
        e = emb_ref[...]
        x0_ref[...] = jnp.dot(oh, e, preferred_element_type=jnp.float32)
        t1 = jnp.dot(e, w1_ref[...], preferred_element_type=jnp.float32)
        y1 = dis * jnp.dot(oh, t1, preferred_element_type=jnp.float32)
        dh = d // 2
        y1_ref[...] = jnp.stack([y1[:, :dh], y1[:, dh:]])

    return pl.pallas_call(
        body,
        grid=(n // nb,),
        in_specs=[pl.BlockSpec((nb, 1), lambda i: (i, 0)),
                  pl.BlockSpec((nb, 1), lambda i: (i, 0)),
                  pl.BlockSpec((v, d), lambda i: (0, 0)),
                  pl.BlockSpec((d, h), lambda i: (0, 0))],
        out_specs=[pl.BlockSpec((nb, 1), lambda i: (i, 0)),
                   pl.BlockSpec((nb, d), lambda i: (i, 0)),
                   pl.BlockSpec((2, nb, h // 2), lambda i: (0, i, 0))],
        out_shape=[jax.ShapeDtypeStruct((n, 1), jnp.float32),
                   jax.ShapeDtypeStruct((n, d), jnp.float32),
                   jax.ShapeDtypeStruct((2, n, h // 2), jnp.float32)],
    )(deg_col, idx_col, emb, W1)


def _mid_stage(x0, s1, dis_col, W2, b1_row, nb):
    n, d = x0.shape
    o = W2.shape[1]
    dh = d // 2

    def body(x0_ref, s_ref, dis_ref, w2_ref, b1_ref, x1_ref, y2_ref):
        dis = dis_ref[...]
        s = s_ref[...]
        seg = jnp.concatenate([s[0], s[1]], axis=1)
        x1 = x0_ref[...] + dis * seg + b1_ref[...]
        x1_ref[...] = x1
        y2 = dis * jnp.dot(x1, w2_ref[...],
                           preferred_element_type=jnp.float32)
        y2_ref[...] = jnp.stack([y2[:, :dh], y2[:, dh:]])

    return pl.pallas_call(
        body,
        grid=(n // nb,),
        in_specs=[pl.BlockSpec((nb, d), lambda i: (i, 0)),
                  pl.BlockSpec((2, nb, dh), lambda i: (0, i, 0)),
                  pl.BlockSpec((nb, 1), lambda i: (i, 0)),
                  pl.BlockSpec((d, o), lambda i: (0, 0)),
                  pl.BlockSpec((1, d), lambda i: (0, 0))],
        out_specs=[pl.BlockSpec((nb, d), lambda i: (i, 0)),
                   pl.BlockSpec((2, nb, o // 2), lambda i: (0, i, 0))],
        out_shape=[jax.ShapeDtypeStruct((n, d), jnp.float32),
                   jax.ShapeDtypeStruct((2, n, o // 2), jnp.float32)],
    )(x0, s1, dis_col, W2, b1_row)


def _final_stage(x1, s2, dis_col, b2_row, nb):
    n, d = x1.shape
    dh = d // 2

    def body(x1_ref, s_ref, dis_ref, b2_ref, out_ref):
        s = s_ref[...]
        seg = jnp.concatenate([s[0], s[1]], axis=1)
        out_ref[...] = x1_ref[...] + dis_ref[...] * seg + b2_ref[...]

    return pl.pallas_call(
        body,
        grid=(n // nb,),
        in_specs=[pl.BlockSpec((nb, d), lambda i: (i, 0)),
                  pl.BlockSpec((2, nb, dh), lambda i: (0, i, 0)),
                  pl.BlockSpec((nb, 1), lambda i: (i, 0)),
                  pl.BlockSpec((1, d), lambda i: (0, 0))],
        out_specs=pl.BlockSpec((nb, d), lambda i: (i, 0)),
        out_shape=jax.ShapeDtypeStruct((n, d), jnp.float32),
    )(x1, s2, dis_col, b2_row)


def _segsum_sc(yh, src_p, dst_p, zeros_blk, acc_rows):
    ch = src_p.shape[1]
    dh = yh.shape[2]
    zpt = acc_rows // NS
    mesh = plsc.VectorSubcoreMesh(core_axis_name="c", subcore_axis_name="s")

    @functools.partial(
        pl.kernel,
        out_type=jax.ShapeDtypeStruct((NC, acc_rows, dh), jnp.float32),
        mesh=mesh,
        scratch_types=[
            pltpu.VMEM((ch, 128), jnp.int32),
            pltpu.VMEM((ch, 128), jnp.int32),
            pltpu.VMEM((128, dh), jnp.float32),
            pltpu.VMEM((128, dh), jnp.float32),
            pltpu.VMEM_SHARED((acc_rows, dh), jnp.float32),
            pltpu.SemaphoreType.DMA,
            pltpu.SemaphoreType.DMA,
        ],
        compiler_params=pltpu.CompilerParams(use_tc_tiling_on_sc=False),
    )
    def k(y_hbm, src_hbm, dst_hbm, z_hbm, out_hbm,
          src_v, dst_v, buf_a, buf_b, acc, sem_a, sem_b):
        c = lax.axis_index("c")
        s = lax.axis_index("s")
        yc = y_hbm.at[c]
        pltpu.sync_copy(z_hbm, acc.at[pl.ds(s * zpt, zpt)])
        pltpu.sync_copy(src_hbm.at[s], src_v)
        pltpu.sync_copy(dst_hbm.at[s], dst_v)
        plsc.subcore_barrier()

        pltpu.async_copy(yc.at[src_v.at[0]], buf_a, sem_a)

        def body(g, carry):
            j0 = 2 * g
            j1 = j0 + 1
            pltpu.async_copy(yc.at[src_v.at[j1]], buf_b, sem_b)
            pltpu.make_async_copy(yc.at[src_v.at[j0]], buf_a, sem_a).wait()
            pltpu.sync_copy(buf_a, acc.at[dst_v.at[j0]], add=True)

            @pl.when(j0 + 2 < ch)
            def _():
                pltpu.async_copy(yc.at[src_v.at[j0 + 2]], buf_a, sem_a)

            pltpu.make_async_copy(yc.at[src_v.at[j1]], buf_b, sem_b).wait()
            pltpu.sync_copy(buf_b, acc.at[dst_v.at[j1]], add=True)
            return carry

        lax.fori_loop(0, ch // 2, body, 0)
        plsc.subcore_barrier()
        pltpu.sync_copy(acc.at[pl.ds(s * zpt, zpt)],
                        out_hbm.at[c].at[pl.ds(s * zpt, zpt)])

    return k(yh, src_p, dst_p, zeros_blk)


def _split_cols(y):
    dh = y.shape[1] // 2
    return jnp.stack([y[:, :dh], y[:, dh:]])


def kernel(phoneme_indices, edge_index, emb, W1, b1, W2, b2):
    n = phoneme_indices.shape[0]
    e = edge_index.shape[1]
    d = emb.shape[1]
    h = W1.shape[1]
    o = W2.shape[1]

    idx_col = phoneme_indices.astype(jnp.int32).reshape(n, 1)
    src = edge_index[0].astype(jnp.int32)
    dst = edge_index[1].astype(jnp.int32)

    hi_rows = -(-(n + 1) // 128)
    hi_rows = -(-hi_rows // 8) * 8
    be = 2000
    nbe = -(-e // be)
    dst_deg = dst
    if nbe * be != e:
        dst_deg = jnp.concatenate(
            [dst, jnp.full((nbe * be - e,), hi_rows * 128 - 1, jnp.int32)])
    m = _degree_counts(dst_deg.reshape(nbe, 1, be), hi_rows)
    deg_col = m.reshape(-1)[:n].reshape(n, 1)

    nb = 1000
    dis_col, x0, y1 = _embed_stage(deg_col, idx_col, emb, W1, nb)

    ch = -(-e // (NS * 128))
    ch = ch + (ch % 2)
    etot = NS * ch * 128
    src_p = jnp.concatenate([src, jnp.zeros((etot - e,), jnp.int32)])
    dst_p = jnp.concatenate([dst, jnp.full((etot - e,), n, jnp.int32)])
    src_p = src_p.reshape(NS, ch, 128)
    dst_p = dst_p.reshape(NS, ch, 128)
    acc_rows = -(-(n + 1) // (NS * 8)) * (NS * 8)
    zeros_blk = jnp.zeros((acc_rows // NS, d // 2), jnp.float32)

    s1 = _segsum_sc(y1, src_p, dst_p, zeros_blk, acc_rows)

    x1, y2 = _mid_stage(x0, s1, dis_col, W2, b1.reshape(1, h), nb)

    s2 = _segsum_sc(y2, src_p, dst_p, zeros_blk, acc_rows)

    return _final_stage(x1, s2, dis_col, b2.reshape(1, o), nb)


# speedup vs baseline: 1.0985x; 1.0985x over previous
import functools

import jax
import jax.numpy as jnp
from jax import lax
from jax.experimental import pallas as pl
from jax.experimental.pallas import tpu as pltpu
from jax.experimental.pallas import tpu_sc as plsc

NC = 2
NS = 16
NW = NC * NS


def _degree_counts(dst3, hi_rows):
    nblk, _, be = dst3.shape

    def body(dst_ref, m_ref):
        @pl.when(pl.program_id(0) == 0)
        def _():
            m_ref[...] = jnp.zeros_like(m_ref)

        d = dst_ref[...].reshape(1, be)
        oh_hi = (d // 128 == lax.broadcasted_iota(jnp.int32, (hi_rows, be), 0)
                 ).astype(jnp.float32)
        oh_lo = (d % 128 == lax.broadcasted_iota(jnp.int32, (128, be), 0)
                 ).astype(jnp.float32)
        m_ref[...] += lax.dot_general(oh_hi, oh_lo, (((1,), (1,)), ((), ())),
                                      preferred_element_type=jnp.float32)

    return pl.pallas_call(
        body,
        grid=(nblk,),
        in_specs=[pl.BlockSpec((1, 1, be), lambda i: (i, 0, 0))],
        out_specs=pl.BlockSpec((hi_rows, 128), lambda i: (0, 0)),
        out_shape=jax.ShapeDtypeStruct((hi_rows, 128), jnp.float32),
    )(dst3)


def _embed_stage(deg_col, idx_col, emb, W1, nb):
    n = deg_col.shape[0]
    v, d = emb.shape
    h = W1.shape[1]

    def body(deg_ref, idx_ref, emb_ref, w1_ref, dis_ref, x0_ref, y1_ref):
        deg = deg_ref[...]
        dis = jnp.where(deg > 0, lax.rsqrt(jnp.where(deg > 0, deg, 1.0)), 0.0)
        dis_ref[...] = dis
        oh = (idx_ref[...] == lax.broadcasted_iota(jnp.int32, (1, v), 1)
              ).astype(jnp.float32)
        e = emb_ref[...]
        x0_ref[...] = jnp.dot(oh, e, preferred_element_type=jnp.float32)
        t1 = jnp.dot(e, w1_ref[...], preferred_element_type=jnp.float32)
        y1 = dis * jnp.dot(oh, t1, preferred_element_type=jnp.float32)
        dh = d // 2
        y1_ref[...] = jnp.stack([y1[:, :dh], y1[:, dh:]])

    return pl.pallas_call(
        body,
        grid=(n // nb,),
        in_specs=[pl.BlockSpec((nb, 1), lambda i: (i, 0)),
                  pl.BlockSpec((nb, 1), lambda i: (i, 0)),
                  pl.BlockSpec((v, d), lambda i: (0, 0)),
                  pl.BlockSpec((d, h), lambda i: (0, 0))],
        out_specs=[pl.BlockSpec((nb, 1), lambda i: (i, 0)),
                   pl.BlockSpec((nb, d), lambda i: (i, 0)),
                   pl.BlockSpec((2, nb, h // 2), lambda i: (0, i, 0))],
        out_shape=[jax.ShapeDtypeStruct((n, 1), jnp.float32),
                   jax.ShapeDtypeStruct((n, d), jnp.float32),
                   jax.ShapeDtypeStruct((2, n, h // 2), jnp.float32)],
    )(deg_col, idx_col, emb, W1)


def _mid_stage(x0, s1, dis_col, W2, b1_row, nb):
    n, d = x0.shape
    o = W2.shape[1]
    dh = d // 2

    def body(x0_ref, s_ref, dis_ref, w2_ref, b1_ref, x1_ref, y2_ref):
        dis = dis_ref[...]
        s = s_ref[...]
        seg = jnp.concatenate([s[0], s[1]], axis=1)
        x1 = x0_ref[...] + dis * seg + b1_ref[...]
        x1_ref[...] = x1
        y2 = dis * jnp.dot(x1, w2_ref[...],
                           preferred_element_type=jnp.float32)
        y2_ref[...] = jnp.stack([y2[:, :dh], y2[:, dh:]])

    return pl.pallas_call(
        body,
        grid=(n // nb,),
        in_specs=[pl.BlockSpec((nb, d), lambda i: (i, 0)),
                  pl.BlockSpec((2, nb, dh), lambda i: (0, i, 0)),
                  pl.BlockSpec((nb, 1), lambda i: (i, 0)),
                  pl.BlockSpec((d, o), lambda i: (0, 0)),
                  pl.BlockSpec((1, d), lambda i: (0, 0))],
        out_specs=[pl.BlockSpec((nb, d), lambda i: (i, 0)),
                   pl.BlockSpec((2, nb, o // 2), lambda i: (0, i, 0))],
        out_shape=[jax.ShapeDtypeStruct((n, d), jnp.float32),
                   jax.ShapeDtypeStruct((2, n, o // 2), jnp.float32)],
    )(x0, s1, dis_col, W2, b1_row)


def _final_stage(x1, s2, dis_col, b2_row, nb):
    n, d = x1.shape
    dh = d // 2

    def body(x1_ref, s_ref, dis_ref, b2_ref, out_ref):
        s = s_ref[...]
        seg = jnp.concatenate([s[0], s[1]], axis=1)
        out_ref[...] = x1_ref[...] + dis_ref[...] * seg + b2_ref[...]

    return pl.pallas_call(
        body,
        grid=(n // nb,),
        in_specs=[pl.BlockSpec((nb, d), lambda i: (i, 0)),
                  pl.BlockSpec((2, nb, dh), lambda i: (0, i, 0)),
                  pl.BlockSpec((nb, 1), lambda i: (i, 0)),
                  pl.BlockSpec((1, d), lambda i: (0, 0))],
        out_specs=pl.BlockSpec((nb, d), lambda i: (i, 0)),
        out_shape=jax.ShapeDtypeStruct((n, d), jnp.float32),
    )(x1, s2, dis_col, b2_row)


def _segsum_sc(yh, src_p, dst_p, zeros_blk, acc_rows):
    ch = src_p.shape[1]
    dh = yh.shape[2]
    zpt = acc_rows // NS
    mesh = plsc.VectorSubcoreMesh(core_axis_name="c", subcore_axis_name="s")

    @functools.partial(
        pl.kernel,
        out_type=jax.ShapeDtypeStruct((NC, acc_rows, dh), jnp.float32),
        mesh=mesh,
        scratch_types=[
            pltpu.VMEM((ch, 128), jnp.int32),
            pltpu.VMEM((ch, 128), jnp.int32),
            pltpu.VMEM((128, dh), jnp.float32),
            pltpu.VMEM((128, dh), jnp.float32),
            pltpu.VMEM_SHARED((acc_rows, dh), jnp.float32),
            pltpu.SemaphoreType.DMA,
            pltpu.SemaphoreType.DMA,
        ],
        compiler_params=pltpu.CompilerParams(use_tc_tiling_on_sc=False),
    )
    def k(y_hbm, src_hbm, dst_hbm, z_hbm, out_hbm,
          src_v, dst_v, buf_a, buf_b, acc, sem_a, sem_b):
        c = lax.axis_index("c")
        s = lax.axis_index("s")
        yc = y_hbm.at[c]
        pltpu.sync_copy(z_hbm, acc.at[pl.ds(s * zpt, zpt)])
        pltpu.sync_copy(src_hbm.at[s], src_v)
        pltpu.sync_copy(dst_hbm.at[s], dst_v)
        plsc.subcore_barrier()

        pltpu.async_copy(yc.at[src_v.at[0]], buf_a, sem_a)

        def body(g, carry):
            j0 = 2 * g
            j1 = j0 + 1
            pltpu.async_copy(yc.at[src_v.at[j1]], buf_b, sem_b)
            pltpu.make_async_copy(yc.at[src_v.at[j0]], buf_a, sem_a).wait()
            pltpu.sync_copy(buf_a, acc.at[dst_v.at[j0]], add=True)

            @pl.when(j0 + 2 < ch)
            def _():
                pltpu.async_copy(yc.at[src_v.at[j0 + 2]], buf_a, sem_a)

            pltpu.make_async_copy(yc.at[src_v.at[j1]], buf_b, sem_b).wait()
            pltpu.sync_copy(buf_b, acc.at[dst_v.at[j1]], add=True)
            return carry

        lax.fori_loop(0, ch // 2, body, 0)
        plsc.subcore_barrier()
        pltpu.sync_copy(acc.at[pl.ds(s * zpt, zpt)],
                        out_hbm.at[c].at[pl.ds(s * zpt, zpt)])

    return k(yh, src_p, dst_p, zeros_blk)


def _split_cols(y):
    dh = y.shape[1] // 2
    return jnp.stack([y[:, :dh], y[:, dh:]])


def kernel(phoneme_indices, edge_index, emb, W1, b1, W2, b2):
    n = phoneme_indices.shape[0]
    e = edge_index.shape[1]
    d = emb.shape[1]
    h = W1.shape[1]
    o = W2.shape[1]

    idx_col = phoneme_indices.astype(jnp.int32).reshape(n, 1)
    src = edge_index[0].astype(jnp.int32)
    dst = edge_index[1].astype(jnp.int32)

    hi_rows = -(-(n + 1) // 128)
    hi_rows = -(-hi_rows // 8) * 8
    be = 4000
    nbe = -(-e // be)
    dst_deg = dst
    if nbe * be != e:
        dst_deg = jnp.concatenate(
            [dst, jnp.full((nbe * be - e,), hi_rows * 128 - 1, jnp.int32)])
    m = _degree_counts(dst_deg.reshape(nbe, 1, be), hi_rows)
    deg_col = m.reshape(-1)[:n].reshape(n, 1)

    nb = 2000
    dis_col, x0, y1 = _embed_stage(deg_col, idx_col, emb, W1, nb)

    ch = -(-e // (NS * 128))
    ch = ch + (ch % 2)
    etot = NS * ch * 128
    src_p = jnp.concatenate([src, jnp.zeros((etot - e,), jnp.int32)])
    dst_p = jnp.concatenate([dst, jnp.full((etot - e,), n, jnp.int32)])
    src_p = src_p.reshape(NS, ch, 128)
    dst_p = dst_p.reshape(NS, ch, 128)
    acc_rows = -(-(n + 1) // (NS * 8)) * (NS * 8)
    zeros_blk = jnp.zeros((acc_rows // NS, d // 2), jnp.float32)

    s1 = _segsum_sc(y1, src_p, dst_p, zeros_blk, acc_rows)

    x1, y2 = _mid_stage(x0, s1, dis_col, W2, b1.reshape(1, h), nb)

    s2 = _segsum_sc(y2, src_p, dst_p, zeros_blk, acc_rows)

    return _final_stage(x1, s2, dis_col, b2.reshape(1, o), nb)


# speedup vs baseline: 1.1398x; 1.0376x over previous
import functools

import jax
import jax.numpy as jnp
from jax import lax
from jax.experimental import pallas as pl
from jax.experimental.pallas import tpu as pltpu
from jax.experimental.pallas import tpu_sc as plsc

NC = 2
NS = 16
NW = NC * NS


def _degree_counts(dst3, hi_rows):
    nblk, _, be = dst3.shape

    def body(dst_ref, m_ref):
        @pl.when(pl.program_id(0) == 0)
        def _():
            m_ref[...] = jnp.zeros_like(m_ref)

        d = dst_ref[...].reshape(1, be)
        oh_hi = (d // 128 == lax.broadcasted_iota(jnp.int32, (hi_rows, be), 0)
                 ).astype(jnp.float32)
        oh_lo = (d % 128 == lax.broadcasted_iota(jnp.int32, (128, be), 0)
                 ).astype(jnp.float32)
        m_ref[...] += lax.dot_general(oh_hi, oh_lo, (((1,), (1,)), ((), ())),
                                      preferred_element_type=jnp.float32)

    return pl.pallas_call(
        body,
        grid=(nblk,),
        in_specs=[pl.BlockSpec((1, 1, be), lambda i: (i, 0, 0))],
        out_specs=pl.BlockSpec((hi_rows, 128), lambda i: (0, 0)),
        out_shape=jax.ShapeDtypeStruct((hi_rows, 128), jnp.float32),
    )(dst3)


def _embed_stage(deg_col, idx_col, emb, W1, nb):
    n = deg_col.shape[0]
    v, d = emb.shape
    h = W1.shape[1]

    def body(deg_ref, idx_ref, emb_ref, w1_ref, dis_ref, x0_ref, y1_ref):
        deg = deg_ref[...]
        dis = jnp.where(deg > 0, lax.rsqrt(jnp.where(deg > 0, deg, 1.0)), 0.0)
        dis_ref[...] = dis
        oh = (idx_ref[...] == lax.broadcasted_iota(jnp.int32, (1, v), 1)
              ).astype(jnp.float32)
        e = emb_ref[...]
        x0_ref[...] = jnp.dot(oh, e, preferred_element_type=jnp.float32)
        t1 = jnp.dot(e, w1_ref[...], preferred_element_type=jnp.float32)
        y1 = dis * jnp.dot(oh, t1, preferred_element_type=jnp.float32)
        dh = d // 2
        y1_ref[...] = jnp.stack([y1[:, :dh], y1[:, dh:]])

    return pl.pallas_call(
        body,
        grid=(n // nb,),
        in_specs=[pl.BlockSpec((nb, 1), lambda i: (i, 0)),
                  pl.BlockSpec((nb, 1), lambda i: (i, 0)),
                  pl.BlockSpec((v, d), lambda i: (0, 0)),
                  pl.BlockSpec((d, h), lambda i: (0, 0))],
        out_specs=[pl.BlockSpec((nb, 1), lambda i: (i, 0)),
                   pl.BlockSpec((nb, d), lambda i: (i, 0)),
                   pl.BlockSpec((2, nb, h // 2), lambda i: (0, i, 0))],
        out_shape=[jax.ShapeDtypeStruct((n, 1), jnp.float32),
                   jax.ShapeDtypeStruct((n, d), jnp.float32),
                   jax.ShapeDtypeStruct((2, n, h // 2), jnp.float32)],
    )(deg_col, idx_col, emb, W1)


def _mid_stage(x0, s1, dis_col, W2, b1_row, nb):
    n, d = x0.shape
    o = W2.shape[1]
    dh = d // 2

    def body(x0_ref, s_ref, dis_ref, w2_ref, b1_ref, x1_ref, y2_ref):
        dis = dis_ref[...]
        s = s_ref[...]
        seg = jnp.concatenate([s[0], s[1]], axis=1)
        x1 = x0_ref[...] + dis * seg + b1_ref[...]
        x1_ref[...] = x1
        y2 = dis * jnp.dot(x1, w2_ref[...],
                           preferred_element_type=jnp.float32)
        y2_ref[...] = jnp.stack([y2[:, :dh], y2[:, dh:]])

    return pl.pallas_call(
        body,
        grid=(n // nb,),
        in_specs=[pl.BlockSpec((nb, d), lambda i: (i, 0)),
                  pl.BlockSpec((2, nb, dh), lambda i: (0, i, 0)),
                  pl.BlockSpec((nb, 1), lambda i: (i, 0)),
                  pl.BlockSpec((d, o), lambda i: (0, 0)),
                  pl.BlockSpec((1, d), lambda i: (0, 0))],
        out_specs=[pl.BlockSpec((nb, d), lambda i: (i, 0)),
                   pl.BlockSpec((2, nb, o // 2), lambda i: (0, i, 0))],
        out_shape=[jax.ShapeDtypeStruct((n, d), jnp.float32),
                   jax.ShapeDtypeStruct((2, n, o // 2), jnp.float32)],
    )(x0, s1, dis_col, W2, b1_row)


def _final_stage(x1, s2, dis_col, b2_row, nb):
    n, d = x1.shape
    dh = d // 2

    def body(x1_ref, s_ref, dis_ref, b2_ref, out_ref):
        s = s_ref[...]
        seg = jnp.concatenate([s[0], s[1]], axis=1)
        out_ref[...] = x1_ref[...] + dis_ref[...] * seg + b2_ref[...]

    return pl.pallas_call(
        body,
        grid=(n // nb,),
        in_specs=[pl.BlockSpec((nb, d), lambda i: (i, 0)),
                  pl.BlockSpec((2, nb, dh), lambda i: (0, i, 0)),
                  pl.BlockSpec((nb, 1), lambda i: (i, 0)),
                  pl.BlockSpec((1, d), lambda i: (0, 0))],
        out_specs=pl.BlockSpec((nb, d), lambda i: (i, 0)),
        out_shape=jax.ShapeDtypeStruct((n, d), jnp.float32),
    )(x1, s2, dis_col, b2_row)


def _segsum_sc(yh, src_p, dst_p, zeros_blk, acc_rows):
    ch = src_p.shape[1]
    dh = yh.shape[2]
    zpt = acc_rows // NS
    mesh = plsc.VectorSubcoreMesh(core_axis_name="c", subcore_axis_name="s")

    @functools.partial(
        pl.kernel,
        out_type=jax.ShapeDtypeStruct((NC, acc_rows, dh), jnp.float32),
        mesh=mesh,
        scratch_types=[
            pltpu.VMEM((ch, 128), jnp.int32),
            pltpu.VMEM((ch, 128), jnp.int32),
            pltpu.VMEM((128, dh), jnp.float32),
            pltpu.VMEM((128, dh), jnp.float32),
            pltpu.VMEM_SHARED((acc_rows, dh), jnp.float32),
            pltpu.SemaphoreType.DMA,
            pltpu.SemaphoreType.DMA,
        ],
        compiler_params=pltpu.CompilerParams(use_tc_tiling_on_sc=False),
    )
    def k(y_hbm, src_hbm, dst_hbm, z_hbm, out_hbm,
          src_v, dst_v, buf_a, buf_b, acc, sem_a, sem_b):
        c = lax.axis_index("c")
        s = lax.axis_index("s")
        yc = y_hbm.at[c]
        pltpu.sync_copy(z_hbm, acc.at[pl.ds(s * zpt, zpt)])
        pltpu.sync_copy(src_hbm.at[s], src_v)
        pltpu.sync_copy(dst_hbm.at[s], dst_v)
        plsc.subcore_barrier()

        pltpu.async_copy(yc.at[src_v.at[0]], buf_a, sem_a)

        def body(g, carry):
            j0 = 2 * g
            j1 = j0 + 1
            pltpu.async_copy(yc.at[src_v.at[j1]], buf_b, sem_b)
            pltpu.make_async_copy(yc.at[src_v.at[j0]], buf_a, sem_a).wait()
            pltpu.sync_copy(buf_a, acc.at[dst_v.at[j0]], add=True)

            @pl.when(j0 + 2 < ch)
            def _():
                pltpu.async_copy(yc.at[src_v.at[j0 + 2]], buf_a, sem_a)

            pltpu.make_async_copy(yc.at[src_v.at[j1]], buf_b, sem_b).wait()
            pltpu.sync_copy(buf_b, acc.at[dst_v.at[j1]], add=True)
            return carry

        lax.fori_loop(0, ch // 2, body, 0)
        plsc.subcore_barrier()
        pltpu.sync_copy(acc.at[pl.ds(s * zpt, zpt)],
                        out_hbm.at[c].at[pl.ds(s * zpt, zpt)])

    return k(yh, src_p, dst_p, zeros_blk)


def _split_cols(y):
    dh = y.shape[1] // 2
    return jnp.stack([y[:, :dh], y[:, dh:]])


def kernel(phoneme_indices, edge_index, emb, W1, b1, W2, b2):
    n = phoneme_indices.shape[0]
    e = edge_index.shape[1]
    d = emb.shape[1]
    h = W1.shape[1]
    o = W2.shape[1]

    idx_col = phoneme_indices.astype(jnp.int32).reshape(n, 1)
    src = edge_index[0].astype(jnp.int32)
    dst = edge_index[1].astype(jnp.int32)

    hi_rows = -(-(n + 1) // 128)
    hi_rows = -(-hi_rows // 8) * 8
    be = 8000
    nbe = -(-e // be)
    dst_deg = dst
    if nbe * be != e:
        dst_deg = jnp.concatenate(
            [dst, jnp.full((nbe * be - e,), hi_rows * 128 - 1, jnp.int32)])
    m = _degree_counts(dst_deg.reshape(nbe, 1, be), hi_rows)
    deg_col = m.reshape(-1)[:n].reshape(n, 1)

    nb = 5000
    dis_col, x0, y1 = _embed_stage(deg_col, idx_col, emb, W1, nb)

    ch = -(-e // (NS * 128))
    ch = ch + (ch % 2)
    etot = NS * ch * 128
    src_p = jnp.concatenate([src, jnp.zeros((etot - e,), jnp.int32)])
    dst_p = jnp.concatenate([dst, jnp.full((etot - e,), n, jnp.int32)])
    src_p = src_p.reshape(NS, ch, 128)
    dst_p = dst_p.reshape(NS, ch, 128)
    acc_rows = -(-(n + 1) // (NS * 8)) * (NS * 8)
    zeros_blk = jnp.zeros((acc_rows // NS, d // 2), jnp.float32)

    s1 = _segsum_sc(y1, src_p, dst_p, zeros_blk, acc_rows)

    x1, y2 = _mid_stage(x0, s1, dis_col, W2, b1.reshape(1, h), nb)

    s2 = _segsum_sc(y2, src_p, dst_p, zeros_blk, acc_rows)

    return _final_stage(x1, s2, dis_col, b2.reshape(1, o), nb)
